# Initial kernel scaffold; baseline (speedup 1.0000x reference)
import jax
import jax.numpy as jnp
from jax.experimental import pallas as pl


def kernel(x, edge_src, edge_dst, edge_sh, edge_rbf, edge_len, ln_w, ln_b, wm_w1, wm_b1, wm_w2, wm_b2, wm_w3, wm_b3, eg_w1, eg_b1, eg_w2, eg_b2, mm_w1, mm_b1, mm_w2, mm_b2, sl_w, sl_b, ul_w, ul_b, res_scale):
    raise NotImplementedError("write your pallas kernel here")


# speedup vs baseline: 1.4753x; 1.4753x over previous
import jax
import jax.numpy as jnp
from jax import lax
from jax.experimental import pallas as pl
from jax.experimental.pallas import tpu as pltpu
from jax.experimental.pallas import tpu_sc as plsc

N = 10000
E = 320000
D = 128
R = 16
H = 128
CUTOFF = 5.0
NP = 10240
NHI = NP // 128
BE = 4000
BN = 2000
BND = 2560
CHUNK = 128
NCHUNKS = E // CHUNK
NTILES = 32


def _dot_t(a, w):
    return lax.dot_general(a, w, (((1,), (1,)), ((), ())),
                           preferred_element_type=jnp.float32)


def _edge_body(rbf_ref, len_ref, dst_ref, wm1, wmb1, wm2, wmb2, wm3, wmb3,
               eg1, egb1, eg2, egb2, rwg_ref, nm_ref):
    rbf = rbf_ref[...]
    g = jax.nn.silu(_dot_t(rbf, eg1[...]) + egb1[...])
    gate = jax.nn.sigmoid(
        jnp.sum(g * eg2[...], axis=1, keepdims=True) + egb2[...])
    r = len_ref[...]
    cc = 0.5 * (jnp.cos((jnp.pi / CUTOFF) * r) + 1.0)
    cc = cc * (r <= CUTOFF).astype(jnp.float32)
    ew = cc * gate
    h = jax.nn.silu(_dot_t(rbf, wm1[...]) + wmb1[...])
    h = jax.nn.silu(_dot_t(h, wm2[...]) + wmb2[...])
    rw = _dot_t(h, wm3[...]) + wmb3[...]
    rwg_ref[...] = rw * ew

    dv = dst_ref[...]
    hi = lax.shift_right_logical(dv, 7)
    lo = lax.bitwise_and(dv, 127)
    ia = lax.broadcasted_iota(jnp.int32, (rbf.shape[0], NHI), 1)
    ib = lax.broadcasted_iota(jnp.int32, (rbf.shape[0], 128), 1)
    a = jnp.where(ia == hi, ew, 0.0)
    b = jnp.where(ib == lo, 1.0, 0.0)
    contrib = lax.dot_general(a, b, (((0,), (0,)), ((), ())),
                              preferred_element_type=jnp.float32)

    @pl.when(pl.program_id(0) == 0)
    def _init():
        nm_ref[...] = jnp.zeros_like(nm_ref)

    nm_ref[...] += contrib


def _ln_body(x_ref, w_ref, b_ref, out_ref):
    xv = x_ref[...]
    mu = jnp.mean(xv, axis=1, keepdims=True)
    var = jnp.mean((xv - mu) ** 2, axis=1, keepdims=True)
    out_ref[...] = (xv - mu) * lax.rsqrt(var + 1e-5) * w_ref[...] + b_ref[...]


def _node_body(mp_ref, nm_ref, xn_ref, x_ref, mm1, mmb1, mm2, mmb2,
               slw, slb, ulw, ulb, rs_ref, out_ref):
    tot = mp_ref[0] + mp_ref[1]
    agg = tot / jnp.maximum(nm_ref[...], 1e-8)
    h = jax.nn.silu(_dot_t(agg, mm1[...]) + mmb1[...])
    ao = _dot_t(h, mm2[...]) + mmb2[...]
    xn = xn_ref[...]
    out = _dot_t(xn, slw[...]) + slb[...] + _dot_t(ao, ulw[...]) + ulb[...]
    out_ref[...] = x_ref[...] + rs_ref[0, 0] * out


def _sc_agg_body(xn_hbm, rwg_hbm, src_hbm, dst_hbm, mparts_hbm,
                 srcb, dstb, gath, rwgb, acc, gsem, rsem):
    c = lax.axis_index("c")
    s = lax.axis_index("s")
    wid = s * 2 + c

    z16 = jnp.zeros((16,), jnp.float32)

    def zrow(i, carry):
        for k in range(D // 16):
            gath[0, i, pl.ds(k * 16, 16)] = z16
        return carry

    lax.fori_loop(0, CHUNK, zrow, 0)
    for t in range(5):
        pltpu.sync_copy(gath.at[0], acc.at[pl.ds(s * 640 + t * CHUNK, CHUNK)])
    plsc.subcore_barrier()

    extra = (wid < (NCHUNKS % NTILES)).astype(jnp.int32)
    n_chunks = NCHUNKS // NTILES + extra

    def chunk_body(j, carry):
        base = (wid + NTILES * j) * CHUNK
        pltpu.sync_copy(src_hbm.at[pl.ds(base, CHUNK)], srcb.at[0])
        pltpu.sync_copy(dst_hbm.at[pl.ds(base, CHUNK)], dstb.at[0])
        cp_g = pltpu.async_copy(xn_hbm.at[srcb.at[0]], gath.at[0], gsem)
        cp_r = pltpu.async_copy(rwg_hbm.at[pl.ds(base, CHUNK)], rwgb.at[0], rsem)
        cp_g.wait()
        cp_r.wait()

        def mrow(i, cc2):
            for k in range(D // 16):
                sl = pl.ds(k * 16, 16)
                gath[0, i, sl] = gath[0, i, sl] * rwgb[0, i, sl]
            return cc2

        lax.fori_loop(0, CHUNK, mrow, 0)
        pltpu.sync_copy(gath.at[0], acc.at[dstb.at[0]], add=True)
        return carry

    lax.fori_loop(0, n_chunks, chunk_body, 0)
    plsc.subcore_barrier()

    for t in range(5):
        row = s * 640 + t * CHUNK
        pltpu.sync_copy(acc.at[pl.ds(row, CHUNK)],
                        mparts_hbm.at[c, pl.ds(row, CHUNK)])


def _full(shape):
    zeros = (0,) * len(shape)
    return pl.BlockSpec(shape, lambda i, z=zeros: z)


def kernel(x, edge_src, edge_dst, edge_sh, edge_rbf, edge_len,
           ln_w, ln_b, wm_w1, wm_b1, wm_w2, wm_b2, wm_w3, wm_b3,
           eg_w1, eg_b1, eg_w2, eg_b2, mm_w1, mm_b1, mm_w2, mm_b2,
           sl_w, sl_b, ul_w, ul_b, res_scale):
    del edge_sh
    f32 = jnp.float32

    rwg, nmat = pl.pallas_call(
        _edge_body,
        grid=(E // BE,),
        in_specs=[
            pl.BlockSpec((BE, R), lambda i: (i, 0)),
            pl.BlockSpec((BE, 1), lambda i: (i, 0)),
            pl.BlockSpec((BE, 1), lambda i: (i, 0)),
            _full((H, R)), _full((1, H)),
            _full((H, H)), _full((1, H)),
            _full((D, H)), _full((1, D)),
            _full((H, R)), _full((1, H)),
            _full((1, H)), _full((1, 1)),
        ],
        out_specs=[
            pl.BlockSpec((BE, D), lambda i: (i, 0)),
            pl.BlockSpec((NHI, 128), lambda i: (0, 0)),
        ],
        out_shape=[
            jax.ShapeDtypeStruct((E, D), f32),
            jax.ShapeDtypeStruct((NHI, 128), f32),
        ],
    )(edge_rbf, edge_len.reshape(E, 1), edge_dst.reshape(E, 1),
      wm_w1, wm_b1.reshape(1, H), wm_w2, wm_b2.reshape(1, H),
      wm_w3, wm_b3.reshape(1, D),
      eg_w1, eg_b1.reshape(1, H), eg_w2, eg_b2.reshape(1, 1))

    xn = pl.pallas_call(
        _ln_body,
        grid=(N // BN,),
        in_specs=[
            pl.BlockSpec((BN, D), lambda i: (i, 0)),
            _full((1, D)), _full((1, D)),
        ],
        out_specs=pl.BlockSpec((BN, D), lambda i: (i, 0)),
        out_shape=jax.ShapeDtypeStruct((N, D), f32),
    )(x, ln_w.reshape(1, D), ln_b.reshape(1, D))

    mesh = plsc.VectorSubcoreMesh(core_axis_name="c", subcore_axis_name="s")
    mparts = pl.kernel(
        _sc_agg_body,
        out_type=jax.ShapeDtypeStruct((2, NP, D), f32),
        mesh=mesh,
        compiler_params=pltpu.CompilerParams(needs_layout_passes=False),
        scratch_types=[
            pltpu.VMEM((1, CHUNK), jnp.int32),
            pltpu.VMEM((1, CHUNK), jnp.int32),
            pltpu.VMEM((1, CHUNK, D), f32),
            pltpu.VMEM((1, CHUNK, D), f32),
            pltpu.VMEM_SHARED((NP, D), f32),
            pltpu.SemaphoreType.DMA,
            pltpu.SemaphoreType.DMA,
        ],
    )(xn, rwg, edge_src, edge_dst)

    out = pl.pallas_call(
        _node_body,
        grid=(pl.cdiv(N, BND),),
        in_specs=[
            pl.BlockSpec((2, BND, D), lambda i: (0, i, 0)),
            pl.BlockSpec((BND, 1), lambda i: (i, 0)),
            pl.BlockSpec((BND, D), lambda i: (i, 0)),
            pl.BlockSpec((BND, D), lambda i: (i, 0)),
            _full((H, D)), _full((1, H)),
            _full((D, H)), _full((1, D)),
            _full((D, D)), _full((1, D)),
            _full((D, D)), _full((1, D)),
            pl.BlockSpec((1, 1), lambda i: (0, 0), memory_space=pltpu.SMEM),
        ],
        out_specs=pl.BlockSpec((BND, D), lambda i: (i, 0)),
        out_shape=jax.ShapeDtypeStruct((N, D), f32),
    )(mparts, nmat.reshape(NP, 1), xn, x,
      mm_w1, mm_b1.reshape(1, H), mm_w2, mm_b2.reshape(1, D),
      sl_w, sl_b.reshape(1, D), ul_w, ul_b.reshape(1, D),
      res_scale.reshape(1, 1))
    return out
